# baseline (speedup 1.0000x reference)
import functools

import jax
import jax.numpy as jnp
from jax import lax
from jax.experimental import pallas as pl
from jax.experimental.pallas import tpu as pltpu
from jax.experimental.pallas import tpu_sc as plsc

def _vshuf(x, idx):
    return lax.gather(
        x, idx[:, None],
        dimension_numbers=lax.GatherDimensionNumbers(
            offset_dims=(), collapsed_slice_dims=(0,), start_index_map=(0,)),
        slice_sizes=(1,), mode=lax.GatherScatterMode.PROMISE_IN_BOUNDS)


N_PATH = 16
K_PATH = 4
L_PATH = 8
DEG = 16

NT = 64
HN = NT // 2
W = NT * N_PATH
HW = HN * N_PATH
NWORKERS = 32


def _sampler_body(n_node, ntiles, nper,
                  nodes_h, neigh_h, cent_h, choices_h, mr_h, out_h,
                  neigh_s, cent_v, nodes_v, choices_v, mr_v,
                  idx_a, idx_b, nxt_a, nxt_b,
                  path_v, score_v, out_v, tbl_v, sem_a, sem_b, sem_c,
                  sem_o, sem_p):
    sid = lax.axis_index("s")
    wid = sid * 2 + lax.axis_index("c")

    seg = n_node * DEG // 16

    @pl.loop(0, seg // 2000)
    def _stage(ci):
        toff = sid * seg + ci * 2000
        pltpu.sync_copy(neigh_h.at[pl.ds(toff, 2000)], tbl_v)
        pltpu.sync_copy(tbl_v, neigh_s.at[pl.ds(toff, 2000)])

    pltpu.sync_copy(cent_h, cent_v)
    plsc.subcore_barrier()

    iota = lax.iota(jnp.int32, 16)
    iota8w = (iota & 7) * W
    lane_lo = iota < 8
    kofn_a = jnp.where(lane_lo, 0, 8 * NT) + (iota & 7) * NT
    kofn_b = kofn_a + 16 * NT
    iota16 = lax.iota(jnp.int32, 16)
    neg_inf = jnp.float32(float("-inf"))
    MRW = 256

    def init_half(lo, idx_ref):
        @pl.loop(lo, lo + HN)
        def _init(v):
            o = v * N_PATH
            ol = o - lo * N_PATH
            start = plsc.load_gather(nodes_v, [jnp.zeros((16,), jnp.int32) + v])
            path_v[pl.ds(o, 16)] = start
            score_v[pl.ds(o, 16)] = plsc.load_gather(cent_v, [start])
            ch = choices_v[0, pl.ds(o, 16)]
            idx_ref[pl.ds(ol, 16)] = ch * n_node + start

    def process_half(lo, nxt_ref, idx_ref, t, off):
        @pl.loop(lo, lo + HN)
        def _step(v):
            o = v * N_PATH
            ol = o - lo * N_PATH
            nxt = nxt_ref[pl.ds(ol, 16)]
            mr = plsc.load_gather(mr_v, [iota16, off + v])
            keep = mr >= t
            path_v[pl.ds(t * W + o, 16)] = jnp.where(keep, nxt, -1)
            c = plsc.load_gather(cent_v, [nxt])
            plsc.addupdate(score_v.at[pl.ds(o, 16)],
                           jnp.where(keep, c, jnp.float32(0.0)))
            if t < L_PATH - 1:
                ch = choices_v[t, pl.ds(o, 16)]
                idx_ref[pl.ds(ol, 16)] = ch * n_node + nxt

    OUTSZ = K_PATH * L_PATH * NT

    def do_tile(tid, parity):
        base = jnp.minimum(tid * NT, n_node - NT)
        wbase = pl.multiple_of(base * N_PATH, 128)
        base0 = pl.multiple_of(base - lax.rem(base, 128), 128)
        off = (base - base0) + jnp.zeros((16,), jnp.int32)
        pltpu.sync_copy(nodes_h.at[pl.ds(base, NT)], nodes_v)
        cp_ch = pltpu.async_copy(choices_h.at[:, pl.ds(wbase, W)],
                                 choices_v, sem_c)
        cp_mr = pltpu.async_copy(mr_h.at[:, pl.ds(base0, MRW)], mr_v, sem_c)
        cp_ch.wait()
        cp_mr.wait()

        init_half(0, idx_a)
        cp_a = pltpu.async_copy(neigh_s.at[idx_a], nxt_a, sem_a)
        init_half(HN, idx_b)
        cp_b = pltpu.async_copy(neigh_s.at[idx_b], nxt_b, sem_b)

        for t in range(1, L_PATH):
            cp_a.wait()
            process_half(0, nxt_a, idx_a, t, off)
            if t < L_PATH - 1:
                cp_a = pltpu.async_copy(neigh_s.at[idx_a], nxt_a, sem_a)
            cp_b.wait()
            process_half(HN, nxt_b, idx_b, t, off)
            if t < L_PATH - 1:
                cp_b = pltpu.async_copy(neigh_s.at[idx_b], nxt_b, sem_b)

        poff = parity * OUTSZ

        @pl.loop(0, NT)
        def _select(v):
            o = v * N_PATH
            s = score_v[pl.ds(o, 16)]
            picks = []
            for _ in range(K_PATH):
                m = s
                for sh in (1, 2, 4, 8):
                    m = jnp.maximum(m, _vshuf(m, iota ^ sh))
                i = plsc.all_reduce_ffs(s == m)
                picks.append(i)
                s = jnp.where(iota == i, neg_inf, s)
            sel_a = jnp.where(lane_lo, picks[0], picks[1])
            pa = plsc.load_gather(path_v, [iota8w + o + sel_a])
            plsc.store_scatter(out_v, [poff + kofn_a + v], pa)
            sel_b = jnp.where(lane_lo, picks[2], picks[3])
            pb = plsc.load_gather(path_v, [iota8w + o + sel_b])
            plsc.store_scatter(out_v, [poff + kofn_b + v], pb)

        @pl.when(parity == 0)
        def _():
            for seg in range(K_PATH * L_PATH):
                pltpu.async_copy(out_v.at[pl.ds(seg * NT, NT)],
                                 out_h.at[pl.ds(seg * n_node + base, NT)],
                                 sem_p)

        @pl.when(parity == 1)
        def _():
            for seg in range(K_PATH * L_PATH):
                pltpu.async_copy(out_v.at[pl.ds(OUTSZ + seg * NT, NT)],
                                 out_h.at[pl.ds(seg * n_node + base, NT)],
                                 sem_o)

    @pl.loop(0, nper)
    def _tiles(j):
        tid = wid + j * NWORKERS
        parity = lax.rem(j, 2)

        @pl.when(tid < ntiles)
        def _():
            @pl.when((j > 1) & (parity == 0))
            def _():
                pltpu.make_async_copy(out_h.at[pl.ds(0, OUTSZ)],
                                      out_v.at[pl.ds(0, OUTSZ)], sem_p).wait()

            @pl.when((j > 1) & (parity == 1))
            def _():
                pltpu.make_async_copy(out_h.at[pl.ds(0, OUTSZ)],
                                      out_v.at[pl.ds(OUTSZ, OUTSZ)],
                                      sem_o).wait()

            do_tile(tid, parity)

    pltpu.make_async_copy(out_h.at[pl.ds(0, OUTSZ)],
                          out_v.at[pl.ds(0, OUTSZ)], sem_p).wait()
    pltpu.make_async_copy(out_h.at[pl.ds(0, OUTSZ)],
                          out_v.at[pl.ds(OUTSZ, OUTSZ)], sem_o).wait()


def kernel(nodes, neighbors, centrality, walk_choices, mask_rand):
    n_node = nodes.shape[0]
    ntiles = -(-n_node // NT)
    nper = -(-ntiles // NWORKERS)
    max_base0 = (n_node - NT) - ((n_node - NT) % 128)
    mr_pad = max_base0 + 256 - n_node

    mesh = plsc.VectorSubcoreMesh(core_axis_name="c", subcore_axis_name="s")
    run = pl.kernel(
        functools.partial(_sampler_body, n_node, ntiles, nper),
        out_type=jax.ShapeDtypeStruct((n_node * K_PATH * L_PATH,), jnp.int32),
        mesh=mesh,
        compiler_params=pltpu.CompilerParams(needs_layout_passes=False),
        scratch_types=[
            pltpu.VMEM_SHARED((n_node * DEG,), jnp.int32),
            pltpu.VMEM((n_node,), jnp.float32),
            pltpu.VMEM((NT,), jnp.int32),
            pltpu.VMEM((L_PATH - 1, W), jnp.int32),
            pltpu.VMEM((N_PATH, 256), jnp.int32),
            pltpu.VMEM((HW,), jnp.int32),
            pltpu.VMEM((HW,), jnp.int32),
            pltpu.VMEM((HW,), jnp.int32),
            pltpu.VMEM((HW,), jnp.int32),
            pltpu.VMEM((L_PATH * W,), jnp.int32),
            pltpu.VMEM((W,), jnp.float32),
            pltpu.VMEM((2 * K_PATH * L_PATH * NT,), jnp.int32),
            pltpu.VMEM((2000,), jnp.int32),
            pltpu.SemaphoreType.DMA,
            pltpu.SemaphoreType.DMA,
            pltpu.SemaphoreType.DMA,
            pltpu.SemaphoreType.DMA,
            pltpu.SemaphoreType.DMA,
        ],
    )
    out = run(nodes,
              jnp.swapaxes(neighbors, 0, 1).reshape(-1),
              centrality,
              jnp.swapaxes(walk_choices, 0, 1),
              jnp.pad(jnp.swapaxes(mask_rand, 0, 1), ((0, 0), (0, mr_pad))))
    return jnp.transpose(out.reshape(K_PATH, L_PATH, n_node), (2, 0, 1))
